# baseline (speedup 1.0000x reference)
import functools

import jax
import jax.numpy as jnp
from jax import lax
from jax.experimental import pallas as pl
from jax.experimental.pallas import tpu as pltpu
from jax.experimental.pallas import tpu_sc as plsc

_N_USERS = 1000000
_N_ITEMS = 100000
_N_OCC = 1000
_EMBED_DIM = 32
_B = 1024


def _sc_gather(feature_table, bias_flat, idx_all):
  info = plsc.get_sparse_core_info()
  nw = info.num_cores * info.num_subcores
  n = idx_all.shape[0]
  per_w = n // nw

  mesh = plsc.VectorSubcoreMesh(core_axis_name="c", subcore_axis_name="s")

  @functools.partial(
      pl.kernel,
      out_type=(
          jax.ShapeDtypeStruct((n, _EMBED_DIM), jnp.float32),
          jax.ShapeDtypeStruct((n,), jnp.float32),
      ),
      mesh=mesh,
      compiler_params=pltpu.CompilerParams(use_tc_tiling_on_sc=False),
      scratch_types=[
          pltpu.VMEM((per_w,), jnp.int32),
          pltpu.VMEM((per_w, _EMBED_DIM), jnp.float32),
          pltpu.VMEM((per_w,), jnp.float32),
          pltpu.SemaphoreType.DMA,
          pltpu.SemaphoreType.DMA,
      ],
  )
  def k(table_hbm, bias_hbm, idx_hbm, rows_out, bias_out,
        idx_v, rows_v, bias_v, sem_r, sem_b):
    wid = lax.axis_index("s") * info.num_cores + lax.axis_index("c")
    base = wid * per_w
    pltpu.sync_copy(idx_hbm.at[pl.ds(base, per_w)], idx_v)
    cp_r = pltpu.async_copy(table_hbm.at[idx_v], rows_v, sem_r)
    cp_b = pltpu.async_copy(bias_hbm.at[idx_v], bias_v, sem_b)
    cp_r.wait()
    cp_b.wait()
    pltpu.sync_copy(rows_v, rows_out.at[pl.ds(base, per_w)])
    pltpu.sync_copy(bias_v, bias_out.at[pl.ds(base, per_w)])

  return k(feature_table, bias_flat, idx_all)


def _tc_body(rows_ref, biasg_ref, out_ref):
  u = rows_ref[0 * _B:1 * _B, :]
  o = rows_ref[1 * _B:2 * _B, :]
  t = rows_ref[2 * _B:3 * _B, :]
  v = rows_ref[3 * _B:4 * _B, :]
  s = u + o + t
  cross = jnp.sum(u * o + u * t + o * t, axis=1)
  row_bias = biasg_ref[0, :] + biasg_ref[1, :] + biasg_ref[2, :]
  item_bias = biasg_ref[3, :]
  inter = lax.dot_general(
      s, v, dimension_numbers=(((1,), (1,)), ((), ())),
      preferred_element_type=jnp.float32)
  out_ref[...] = inter + (cross + row_bias)[:, None] + item_bias[None, :]


def kernel(user_code, item_code, user_occupation, item_timestamp_rank,
           feature_table, bias_table):
  u = user_code.astype(jnp.int32)
  i = item_code.astype(jnp.int32) + _N_USERS
  o = user_occupation.astype(jnp.int32) + (_N_USERS + _N_ITEMS)
  t = item_timestamp_rank.astype(jnp.int32) + (_N_USERS + _N_ITEMS + _N_OCC)
  idx_all = jnp.concatenate([u, o, t, i])
  bias_flat = bias_table[:, 0]

  rows, bias_g = _sc_gather(feature_table, bias_flat, idx_all)

  return pl.pallas_call(
      _tc_body,
      out_shape=jax.ShapeDtypeStruct((_B, _B), jnp.float32),
  )(rows, bias_g.reshape(4, _B))
